# baseline (speedup 1.0000x reference)
import functools

import jax
import jax.numpy as jnp
from jax import lax
from jax.experimental import pallas as pl
from jax.experimental.pallas import tpu as pltpu
from jax.experimental.pallas import tpu_sc as plsc

BATCH = 4096
SEQ = 200
EMBED_DIM = 64
LANES = 16
NUM_WORKERS = 32
ROWS_PER_W = BATCH // NUM_WORKERS
CHUNK = 48
NCHUNK_MAX = 5
IDS_PAD = 248


NBUF = 4


def _body(ids_hbm, lens_hbm, table_hbm, out_hbm, ids_v, lens_v, rows_v,
          out_v, sem0, sem1, sem2, sem3):
    cid = lax.axis_index("c")
    sid = lax.axis_index("s")
    wid = sid * 2 + cid
    base = wid * ROWS_PER_W
    sems = (sem0, sem1, sem2, sem3)

    zeros = jnp.zeros((LANES,), jnp.int32)

    def memset_row(r, _):
        for t in range(3):
            ids_v[r, pl.ds(200 + t * LANES, LANES)] = zeros
        return 0

    lax.fori_loop(0, ROWS_PER_W, memset_row, 0)

    pltpu.sync_copy(ids_hbm.at[pl.ds(base, ROWS_PER_W), :],
                    ids_v.at[:, pl.ds(0, SEQ)])
    pltpu.sync_copy(lens_hbm.at[pl.ds(base, ROWS_PER_W)],
                    lens_v.at[pl.ds(0, ROWS_PER_W)])

    def nchunks(b):
        ln = lens_v[pl.ds(b, LANES)][0]
        return ln, lax.div(ln + (CHUNK - 1), CHUNK)

    def fire(b, buf):
        _, nch = nchunks(b)

        def chunk(c, _):
            off = c * CHUNK
            pltpu.async_copy(
                table_hbm.at[ids_v.at[b, pl.ds(off, CHUNK)]],
                rows_v.at[buf, pl.ds(off, CHUNK), :],
                sems[buf],
            )
            return 0

        lax.fori_loop(0, nch, chunk, 0)

    def drain_sum(b, buf):
        ln, nch = nchunks(b)

        def dchunk(c, _):
            off = c * CHUNK
            pltpu.make_async_copy(
                table_hbm.at[ids_v.at[b, pl.ds(off, CHUNK)]],
                rows_v.at[buf, pl.ds(off, CHUNK), :],
                sems[buf],
            ).wait()
            return 0

        lax.fori_loop(0, nch, dchunk, 0)

        def accum(s, acc):
            return tuple(
                acc[l] + rows_v[buf, s, pl.ds(l * LANES, LANES)]
                for l in range(4)
            )

        acc0 = tuple(jnp.zeros((LANES,), jnp.float32) for _ in range(4))
        acc = lax.fori_loop(0, ln, accum, acc0)

        den = jnp.full((LANES,), lax.max(ln, 1), jnp.int32).astype(jnp.float32)
        for l in range(4):
            out_v[b, pl.ds(l * LANES, LANES)] = acc[l] / den

    for j in range(NBUF):
        fire(j, j)

    def group(i, _):
        b0 = NBUF * i
        for j in range(NBUF):
            b = b0 + j
            drain_sum(b, j)

            @pl.when(b + NBUF < ROWS_PER_W)
            def _():
                fire(b + NBUF, j)

        return 0

    lax.fori_loop(0, ROWS_PER_W // NBUF, group, 0)

    pltpu.sync_copy(out_v, out_hbm.at[pl.ds(base, ROWS_PER_W), :])


@jax.jit
def _pooled(token_ids, token_lens, table):
    mesh = plsc.VectorSubcoreMesh(core_axis_name="c", subcore_axis_name="s")
    f = functools.partial(
        pl.kernel,
        mesh=mesh,
        compiler_params=pltpu.CompilerParams(use_tc_tiling_on_sc=False),
        out_type=jax.ShapeDtypeStruct((BATCH, EMBED_DIM), jnp.float32),
        scratch_types=[
            pltpu.VMEM((ROWS_PER_W, IDS_PAD), jnp.int32),
            pltpu.VMEM((ROWS_PER_W + LANES,), jnp.int32),
            pltpu.VMEM((NBUF, NCHUNK_MAX * CHUNK, EMBED_DIM), jnp.float32),
            pltpu.VMEM((ROWS_PER_W, EMBED_DIM), jnp.float32),
            pltpu.SemaphoreType.DMA,
            pltpu.SemaphoreType.DMA,
            pltpu.SemaphoreType.DMA,
            pltpu.SemaphoreType.DMA,
        ],
    )(_body)
    return f(token_ids, token_lens, table)


def kernel(token_ids, token_lens, table):
    return _pooled(token_ids, token_lens, table)


# speedup vs baseline: 1.1234x; 1.1234x over previous
import functools

import jax
import jax.numpy as jnp
from jax import lax
from jax.experimental import pallas as pl
from jax.experimental.pallas import tpu as pltpu
from jax.experimental.pallas import tpu_sc as plsc

BATCH = 4096
SEQ = 200
EMBED_DIM = 64
PAD_DIM = 128
LANES = 16
NUM_WORKERS = 32
ROWS_PER_W = BATCH // NUM_WORKERS
CHUNK = 48
LAST_OFF = SEQ - CHUNK
NBUF = 2


def _body(ids_hbm, lens_hbm, table_hbm, out_hbm, ids_v, lens_v, rows_v,
          out_v, sem0, sem1):
    cid = lax.axis_index("c")
    sid = lax.axis_index("s")
    wid = sid * 2 + cid
    base = wid * ROWS_PER_W
    sems = (sem0, sem1)

    pltpu.sync_copy(ids_hbm.at[pl.ds(base * SEQ, ROWS_PER_W * SEQ)], ids_v)
    pltpu.sync_copy(lens_hbm.at[pl.ds(base, ROWS_PER_W)],
                    lens_v.at[pl.ds(0, ROWS_PER_W)])

    lane_iota = lax.iota(jnp.int32, LANES)

    def nchunks(b):
        ln = lens_v[pl.ds(b, LANES)][0]
        return ln, lax.div(ln + (CHUNK - 1), CHUNK)

    def fire(b, buf):
        _, nch = nchunks(b)

        def chunk(c, _):
            off = lax.min(c * CHUNK, LAST_OFF)
            pltpu.async_copy(
                table_hbm.at[ids_v.at[pl.ds(b * SEQ + off, CHUNK)]],
                rows_v.at[buf, pl.ds(off, CHUNK), :],
                sems[buf],
            )
            return 0

        lax.fori_loop(0, nch, chunk, 0)

    def drain_sum(b, buf):
        ln, nch = nchunks(b)

        def dchunk(c, _):
            off = lax.min(c * CHUNK, LAST_OFF)
            pltpu.make_async_copy(
                table_hbm.at[ids_v.at[pl.ds(b * SEQ + off, CHUNK)]],
                rows_v.at[buf, pl.ds(off, CHUNK), :],
                sems[buf],
            ).wait()
            return 0

        lax.fori_loop(0, nch, dchunk, 0)

        def accum(s, acc):
            svec = jnp.full((LANES,), s, jnp.int32)
            return tuple(
                acc[l] + plsc.load_gather(
                    rows_v.at[buf], [svec, lane_iota + l * LANES])
                for l in range(4)
            )

        acc0 = tuple(jnp.zeros((LANES,), jnp.float32) for _ in range(4))
        acc = lax.fori_loop(0, ln, accum, acc0)

        den = jnp.full((LANES,), lax.max(ln, 1), jnp.int32).astype(jnp.float32)
        for l in range(4):
            out_v[pl.ds(b * EMBED_DIM + l * LANES, LANES)] = acc[l] / den

    for j in range(NBUF):
        fire(j, j)

    def group(i, _):
        b0 = NBUF * i
        for j in range(NBUF):
            b = b0 + j
            drain_sum(b, j)

            @pl.when(b + NBUF < ROWS_PER_W)
            def _():
                fire(b + NBUF, j)

        return 0

    lax.fori_loop(0, ROWS_PER_W // NBUF, group, 0)

    pltpu.sync_copy(out_v,
                    out_hbm.at[pl.ds(base * EMBED_DIM,
                                     ROWS_PER_W * EMBED_DIM)])


@jax.jit
def _pooled(token_ids, token_lens, table):
    ids_flat = token_ids.reshape(BATCH * SEQ)
    table_pad = jnp.pad(table, ((0, 0), (0, PAD_DIM - EMBED_DIM)))
    mesh = plsc.VectorSubcoreMesh(core_axis_name="c", subcore_axis_name="s")
    f = functools.partial(
        pl.kernel,
        mesh=mesh,
        compiler_params=pltpu.CompilerParams(use_tc_tiling_on_sc=True,
                                             needs_layout_passes=False),
        out_type=jax.ShapeDtypeStruct((BATCH * EMBED_DIM,), jnp.float32),
        scratch_types=[
            pltpu.VMEM((ROWS_PER_W * SEQ,), jnp.int32),
            pltpu.VMEM((ROWS_PER_W + LANES,), jnp.int32),
            pltpu.VMEM((NBUF, SEQ, PAD_DIM), jnp.float32),
            pltpu.VMEM((ROWS_PER_W * EMBED_DIM,), jnp.float32),
            pltpu.SemaphoreType.DMA,
            pltpu.SemaphoreType.DMA,
        ],
    )(_body)
    flat = f(ids_flat, token_lens, table_pad)
    return flat.reshape(BATCH, EMBED_DIM)


def kernel(token_ids, token_lens, table):
    return _pooled(token_ids, token_lens, table)
